# baseline (speedup 1.0000x reference)
import jax
import jax.numpy as jnp
from jax.experimental import pallas as pl

_TILE = 512


def _moe_kernel(x_ref, w1_ref, b1_ref, w2_ref, b2_ref, wg_ref, bg_ref, o_ref):
    E = b2_ref.shape[0]
    H = w1_ref.shape[1] // E
    x = x_ref[...]

    logits = jax.lax.dot_general(
        x, wg_ref[...], (((1,), (0,)), ((), ()))) + bg_ref[...]
    m = jnp.max(logits, axis=1, keepdims=True)
    ex = jnp.exp(logits - m)
    probs = ex / jnp.sum(ex, axis=1, keepdims=True)

    iota = jax.lax.broadcasted_iota(jnp.int32, probs.shape, 1)
    m1 = jnp.max(probs, axis=1, keepdims=True)
    i1 = jnp.min(jnp.where(probs == m1, iota, 127), axis=1, keepdims=True)
    sel1 = iota == i1
    p2 = jnp.where(sel1, -1.0, probs)
    m2 = jnp.max(p2, axis=1, keepdims=True)
    i2 = jnp.min(jnp.where(p2 == m2, iota, 127), axis=1, keepdims=True)
    sel2 = iota == i2
    w = jnp.where(sel1, m1, 0.0) + jnp.where(sel2, m2, 0.0)

    xb = x.astype(jnp.bfloat16)
    h1 = jax.lax.dot_general(
        xb, w1_ref[...].astype(jnp.bfloat16), (((1,), (0,)), ((), ())),
        preferred_element_type=jnp.float32)
    h = jnp.maximum(h1 + b1_ref[...], 0.0)

    rows = jax.lax.broadcasted_iota(jnp.int32, (E, E * H), 0)
    cols = jax.lax.broadcasted_iota(jnp.int32, (E, E * H), 1)
    expand = (rows == cols // H).astype(jnp.float32)
    wexp = jax.lax.dot_general(
        w, expand, (((1,), (0,)), ((), ())),
        precision=jax.lax.Precision.HIGHEST)
    hs = (h * wexp).astype(jnp.bfloat16)
    out = jax.lax.dot_general(
        hs, w2_ref[...].astype(jnp.bfloat16), (((1,), (0,)), ((), ())),
        preferred_element_type=jnp.float32)
    out = out + jax.lax.dot_general(
        w, b2_ref[...], (((1,), (0,)), ((), ())),
        precision=jax.lax.Precision.HIGHEST)
    o_ref[...] = out


def kernel(x, W1, b1, W2, b2, Wg, bg):
    B, S, D = x.shape
    E, _, H = W1.shape
    O = W2.shape[2]
    N = B * S
    xf = x.reshape(N, D)
    W1r = W1.transpose(1, 0, 2).reshape(D, E * H)
    b1r = b1.reshape(1, E * H)
    W2r = W2.reshape(E * H, O)
    bgr = bg.reshape(1, E)
    out = pl.pallas_call(
        _moe_kernel,
        grid=(N // _TILE,),
        in_specs=[
            pl.BlockSpec((_TILE, D), lambda i: (i, 0)),
            pl.BlockSpec((D, E * H), lambda i: (0, 0)),
            pl.BlockSpec((1, E * H), lambda i: (0, 0)),
            pl.BlockSpec((E * H, O), lambda i: (0, 0)),
            pl.BlockSpec((E, O), lambda i: (0, 0)),
            pl.BlockSpec((D, E), lambda i: (0, 0)),
            pl.BlockSpec((1, E), lambda i: (0, 0)),
        ],
        out_specs=pl.BlockSpec((_TILE, O), lambda i: (i, 0)),
        out_shape=jax.ShapeDtypeStruct((N, O), jnp.float32),
    )(xf, W1r, b1r, W2r, b2, Wg, bgr)
    return out.reshape(B, S, O)


# speedup vs baseline: 1.0919x; 1.0919x over previous
import jax
import jax.numpy as jnp
from jax.experimental import pallas as pl

_TILE = 512


def _moe_kernel(x_ref, w1_ref, b1_ref, w2_ref, b2_ref, wg_ref, bg_ref, o_ref):
    E = b2_ref.shape[0]
    H = w1_ref.shape[1] // E
    x = x_ref[...]

    logits = jax.lax.dot_general(
        x, wg_ref[...], (((1,), (0,)), ((), ()))) + bg_ref[...]
    m = jnp.max(logits, axis=1, keepdims=True)
    ex = jnp.exp(logits - m)
    probs = ex / jnp.sum(ex, axis=1, keepdims=True)

    iota = jax.lax.broadcasted_iota(jnp.int32, probs.shape, 1)
    m1 = jnp.max(probs, axis=1, keepdims=True)
    i1 = jnp.min(jnp.where(probs == m1, iota, 127), axis=1, keepdims=True)
    sel1 = iota == i1
    p2 = jnp.where(sel1, -1.0, probs)
    m2 = jnp.max(p2, axis=1, keepdims=True)
    i2 = jnp.min(jnp.where(p2 == m2, iota, 127), axis=1, keepdims=True)
    sel2 = iota == i2
    w = jnp.where(sel1, m1, 0.0) + jnp.where(sel2, m2, 0.0)

    xb = x.astype(jnp.bfloat16)
    h1 = jax.lax.dot_general(
        xb, w1_ref[...].astype(jnp.bfloat16), (((1,), (0,)), ((), ())),
        preferred_element_type=jnp.float32)
    h = jnp.maximum(h1 + b1_ref[...], 0.0)

    hs = jnp.concatenate(
        [h[:, e * H:(e + 1) * H] * w[:, e:e + 1] for e in range(E)],
        axis=1).astype(jnp.bfloat16)
    out = jax.lax.dot_general(
        hs, w2_ref[...].astype(jnp.bfloat16), (((1,), (0,)), ((), ())),
        preferred_element_type=jnp.float32)
    out = out + jax.lax.dot_general(
        w.astype(jnp.bfloat16), b2_ref[...].astype(jnp.bfloat16),
        (((1,), (0,)), ((), ())), preferred_element_type=jnp.float32)
    o_ref[...] = out


def kernel(x, W1, b1, W2, b2, Wg, bg):
    B, S, D = x.shape
    E, _, H = W1.shape
    O = W2.shape[2]
    N = B * S
    xf = x.reshape(N, D)
    W1r = W1.transpose(1, 0, 2).reshape(D, E * H)
    b1r = b1.reshape(1, E * H)
    W2r = W2.reshape(E * H, O)
    bgr = bg.reshape(1, E)
    out = pl.pallas_call(
        _moe_kernel,
        grid=(N // _TILE,),
        in_specs=[
            pl.BlockSpec((_TILE, D), lambda i: (i, 0)),
            pl.BlockSpec((D, E * H), lambda i: (0, 0)),
            pl.BlockSpec((1, E * H), lambda i: (0, 0)),
            pl.BlockSpec((E * H, O), lambda i: (0, 0)),
            pl.BlockSpec((E, O), lambda i: (0, 0)),
            pl.BlockSpec((D, E), lambda i: (0, 0)),
            pl.BlockSpec((1, E), lambda i: (0, 0)),
        ],
        out_specs=pl.BlockSpec((_TILE, O), lambda i: (i, 0)),
        out_shape=jax.ShapeDtypeStruct((N, O), jnp.float32),
    )(xf, W1r, b1r, W2r, b2, Wg, bgr)
    return out.reshape(B, S, O)


# speedup vs baseline: 1.3818x; 1.2655x over previous
import jax
import jax.numpy as jnp
from jax.experimental import pallas as pl

_TILE = 512


def _moe_kernel(x_ref, w1_ref, b1_ref, w2_ref, b2_ref, wg_ref, bg_ref, o_ref):
    E, _, H = w1_ref.shape
    x = x_ref[...]

    logits = jax.lax.dot_general(
        x, wg_ref[...], (((1,), (0,)), ((), ()))) + bg_ref[...]
    m = jnp.max(logits, axis=1, keepdims=True)
    ex = jnp.exp(logits - m)
    probs = ex / jnp.sum(ex, axis=1, keepdims=True)

    iota = jax.lax.broadcasted_iota(jnp.int32, probs.shape, 1)
    m1 = jnp.max(probs, axis=1, keepdims=True)
    i1 = jnp.min(jnp.where(probs == m1, iota, 127), axis=1, keepdims=True)
    sel1 = iota == i1
    p2 = jnp.where(sel1, -1.0, probs)
    m2 = jnp.max(p2, axis=1, keepdims=True)
    i2 = jnp.min(jnp.where(p2 == m2, iota, 127), axis=1, keepdims=True)
    sel2 = iota == i2
    w = jnp.where(sel1, m1, 0.0) + jnp.where(sel2, m2, 0.0)

    xb = x.astype(jnp.bfloat16)
    acc = jax.lax.dot_general(
        w.astype(jnp.bfloat16), b2_ref[...],
        (((1,), (0,)), ((), ())), preferred_element_type=jnp.float32)
    for e in range(E):
        h = jnp.maximum(
            jax.lax.dot_general(
                xb, w1_ref[e], (((1,), (0,)), ((), ())),
                preferred_element_type=jnp.float32) + b1_ref[e:e + 1, :],
            0.0)
        hs = (h * w[:, e:e + 1]).astype(jnp.bfloat16)
        acc = acc + jax.lax.dot_general(
            hs, w2_ref[e], (((1,), (0,)), ((), ())),
            preferred_element_type=jnp.float32)
    o_ref[...] = acc


def kernel(x, W1, b1, W2, b2, Wg, bg):
    B, S, D = x.shape
    E, _, H = W1.shape
    O = W2.shape[2]
    N = B * S
    xf = x.reshape(N, D)
    W1b = W1.astype(jnp.bfloat16)
    W2b = W2.astype(jnp.bfloat16)
    b2b = b2.astype(jnp.bfloat16)
    bgr = bg.reshape(1, E)
    out = pl.pallas_call(
        _moe_kernel,
        grid=(N // _TILE,),
        in_specs=[
            pl.BlockSpec((_TILE, D), lambda i: (i, 0)),
            pl.BlockSpec((E, D, H), lambda i: (0, 0, 0)),
            pl.BlockSpec((E, H), lambda i: (0, 0)),
            pl.BlockSpec((E, H, O), lambda i: (0, 0, 0)),
            pl.BlockSpec((E, O), lambda i: (0, 0)),
            pl.BlockSpec((D, E), lambda i: (0, 0)),
            pl.BlockSpec((1, E), lambda i: (0, 0)),
        ],
        out_specs=pl.BlockSpec((_TILE, O), lambda i: (i, 0)),
        out_shape=jax.ShapeDtypeStruct((N, O), jnp.float32),
    )(xf, W1b, b1, W2b, b2b, Wg, bgr)
    return out.reshape(B, S, O)
